# baseline (speedup 1.0000x reference)
import functools

import jax
import jax.numpy as jnp
from jax import lax
from jax.experimental import pallas as pl
from jax.experimental.pallas import tpu as pltpu
from jax.experimental.pallas import tpu_sc as plsc

VOCAB = 1000000
EMBED_DIM = 64
INPUT_SIZE = 20
TARGET_DIM = 10
BATCH = 16384

N_ROWS = BATCH * INPUT_SIZE


def _make_sc_gather():
    info = plsc.get_sparse_core_info()
    NC, NS = info.num_cores, info.num_subcores
    NW = NC * NS
    rows_per_w = N_ROWS // NW
    CH = 1024
    NCH = rows_per_w // CH

    mesh = plsc.VectorSubcoreMesh(core_axis_name="c", subcore_axis_name="s")

    @functools.partial(
        pl.kernel,
        mesh=mesh,
        out_type=jax.ShapeDtypeStruct((N_ROWS, EMBED_DIM), jnp.float32),
        compiler_params=pltpu.CompilerParams(use_tc_tiling_on_sc=False),
        scratch_types=[
            pltpu.VMEM((CH,), jnp.int32),
            pltpu.VMEM((CH, EMBED_DIM), jnp.float32),
            pltpu.SemaphoreType.DMA,
        ],
    )
    def gather_k(table_hbm, idx_hbm, out_hbm, idx_v, rows_v, sem):
        wid = lax.axis_index("s") * NC + lax.axis_index("c")
        base = wid * rows_per_w

        def body(i, carry):
            off = base + i * CH
            pltpu.sync_copy(idx_hbm.at[pl.ds(off, CH)], idx_v)
            pltpu.async_copy(table_hbm.at[idx_v], rows_v, sem).wait()
            pltpu.sync_copy(rows_v, out_hbm.at[pl.ds(off, CH)])
            return carry

        lax.fori_loop(0, NCH, body, 0)

    return gather_k


_sc_gather = _make_sc_gather()


def _tc_body(f_ref, w_ref, b_ref, o_ref):
    f = jnp.maximum(f_ref[...], 0.0)
    acc = lax.dot_general(
        f, w_ref[...], (((1,), (1,)), ((), ())),
        preferred_element_type=jnp.float32)
    o_ref[...] = acc + b_ref[...]


def _tc_linear(features, W, b2):
    BLK = 512
    grid = (BATCH // BLK,)
    return pl.pallas_call(
        _tc_body,
        grid=grid,
        in_specs=[
            pl.BlockSpec((BLK, INPUT_SIZE * EMBED_DIM), lambda i: (i, 0)),
            pl.BlockSpec((TARGET_DIM, INPUT_SIZE * EMBED_DIM), lambda i: (0, 0)),
            pl.BlockSpec((1, TARGET_DIM), lambda i: (0, 0)),
        ],
        out_specs=pl.BlockSpec((BLK, TARGET_DIM), lambda i: (i, 0)),
        out_shape=jax.ShapeDtypeStruct((BATCH, TARGET_DIM), jnp.float32),
    )(features, W, b2)


def kernel(x, embedding, W, b):
    idx = x.reshape(-1).astype(jnp.int32)
    feats = _sc_gather(embedding, idx)
    f2 = feats.reshape(BATCH, INPUT_SIZE * EMBED_DIM)
    return _tc_linear(f2, W, b.reshape(1, TARGET_DIM))


# speedup vs baseline: 1.0094x; 1.0094x over previous
import functools

import jax
import jax.numpy as jnp
from jax import lax
from jax.experimental import pallas as pl
from jax.experimental.pallas import tpu as pltpu
from jax.experimental.pallas import tpu_sc as plsc

VOCAB = 1000000
EMBED_DIM = 64
INPUT_SIZE = 20
TARGET_DIM = 10
BATCH = 16384

N_ROWS = BATCH * INPUT_SIZE


def _make_sc_gather():
    info = plsc.get_sparse_core_info()
    NC, NS = info.num_cores, info.num_subcores
    NW = NC * NS
    rows_per_w = N_ROWS // NW
    CH = 640
    NCH = rows_per_w // CH

    mesh = plsc.VectorSubcoreMesh(core_axis_name="c", subcore_axis_name="s")

    @functools.partial(
        pl.kernel,
        mesh=mesh,
        out_type=jax.ShapeDtypeStruct((N_ROWS, EMBED_DIM), jnp.float32),
        compiler_params=pltpu.CompilerParams(use_tc_tiling_on_sc=False),
        scratch_types=[
            pltpu.VMEM((rows_per_w,), jnp.int32),
            pltpu.VMEM((CH, EMBED_DIM), jnp.float32),
            pltpu.VMEM((CH, EMBED_DIM), jnp.float32),
            pltpu.SemaphoreType.DMA,
            pltpu.SemaphoreType.DMA,
            pltpu.SemaphoreType.DMA,
            pltpu.SemaphoreType.DMA,
        ],
    )
    def gather_k(table_hbm, idx_hbm, out_hbm, idx_v, rows0, rows1,
                 sg0, sg1, sw0, sw1):
        wid = lax.axis_index("s") * NC + lax.axis_index("c")
        base = wid * rows_per_w
        pltpu.sync_copy(idx_hbm.at[pl.ds(base, rows_per_w)], idx_v)

        rows = (rows0, rows1)
        sg = (sg0, sg1)
        sw = (sw0, sw1)
        cp_g = [None, None]
        cp_w = [None, None]

        def start_gather(i):
            s = i % 2
            cp_g[s] = pltpu.async_copy(
                table_hbm.at[idx_v.at[pl.ds(i * CH, CH)]], rows[s], sg[s])

        start_gather(0)
        for i in range(NCH):
            s = i % 2
            if i + 1 < NCH:
                if cp_w[1 - s] is not None:
                    cp_w[1 - s].wait()
                start_gather(i + 1)
            cp_g[s].wait()
            cp_w[s] = pltpu.async_copy(
                rows[s], out_hbm.at[pl.ds(base + i * CH, CH)], sw[s])
        cp_w[0].wait()
        cp_w[1].wait()

    return gather_k


_sc_gather = _make_sc_gather()


def _tc_body(f_ref, w_ref, b_ref, o_ref):
    f = jnp.maximum(f_ref[...], 0.0)
    acc = lax.dot_general(
        f, w_ref[...], (((1,), (1,)), ((), ())),
        preferred_element_type=jnp.float32)
    o_ref[...] = acc + b_ref[...]


def _tc_linear(features, W, b2):
    BLK = 512
    grid = (BATCH // BLK,)
    return pl.pallas_call(
        _tc_body,
        grid=grid,
        in_specs=[
            pl.BlockSpec((BLK, INPUT_SIZE * EMBED_DIM), lambda i: (i, 0)),
            pl.BlockSpec((TARGET_DIM, INPUT_SIZE * EMBED_DIM), lambda i: (0, 0)),
            pl.BlockSpec((1, TARGET_DIM), lambda i: (0, 0)),
        ],
        out_specs=pl.BlockSpec((BLK, TARGET_DIM), lambda i: (i, 0)),
        out_shape=jax.ShapeDtypeStruct((BATCH, TARGET_DIM), jnp.float32),
    )(features, W, b2)


def kernel(x, embedding, W, b):
    idx = x.reshape(-1).astype(jnp.int32)
    feats = _sc_gather(embedding, idx)
    f2 = feats.reshape(BATCH, INPUT_SIZE * EMBED_DIM)
    return _tc_linear(f2, W, b.reshape(1, TARGET_DIM))
